# baseline (speedup 1.0000x reference)
import functools

import jax
import jax.numpy as jnp
from jax import lax
from jax.experimental import pallas as pl
from jax.experimental.pallas import tpu as pltpu
from jax.experimental.pallas import tpu_sc as plsc

N = 10000
NP = 10240
E = 160000
IN_DIM = 256
HID = 512
CH = 128

NC = 2
NS = 16
EB = 128
EP = 163840
ROWS_T = NP // NS

R = 512
GI = NP // R

_MESH = dict(core_axis_name="c", subcore_axis_name="s", num_cores=NC,
             num_subcores=NS)


def _deg_body(dst_hbm, ones_hbm, zeros_hbm, out_hbm, idx_v, ones_v, acc_sh):
    c = lax.axis_index("c")
    s = lax.axis_index("s")
    w = c * NS + s
    pltpu.sync_copy(ones_hbm, ones_v)
    pltpu.sync_copy(dst_hbm.at[w], idx_v)
    rows = pl.ds(s * ROWS_T, ROWS_T)
    pltpu.sync_copy(zeros_hbm, acc_sh.at[rows])
    plsc.subcore_barrier()

    def step(j, carry):
        pltpu.sync_copy(ones_v, acc_sh.at[idx_v.at[j]], add=True)
        return carry

    lax.fori_loop(0, EP // (NC * NS * EB), step, 0)
    plsc.subcore_barrier()
    pltpu.sync_copy(acc_sh.at[rows], out_hbm.at[c].at[rows])


@jax.jit
def _deg_call(dst4, ones, zeros):
    return pl.kernel(
        _deg_body,
        out_type=jax.ShapeDtypeStruct((NC, NP, CH), jnp.float32),
        mesh=plsc.VectorSubcoreMesh(**_MESH),
        scratch_types=[
            pltpu.VMEM((EP // (NC * NS * EB), EB), jnp.int32),
            pltpu.VMEM((EB, CH), jnp.float32),
            pltpu.VMEM_SHARED((NP, CH), jnp.float32),
        ],
    )(dst4, ones, zeros)


NBUF = 2
NH = 2
NB = EP // (NS * NH * EB)


def _agg_body(src_hbm, dst_hbm, g_hbm, out_hbm, src_v, dst_v, bufs, acc_sh,
              sem_g, sem_s, *, nch):
    c = lax.axis_index("c")
    s = lax.axis_index("s")
    per_core = nch // NC
    rows = pl.ds(s * ROWS_T, ROWS_T)
    for k in range(per_core):
        ch = c * per_core + k
        g_chunk = g_hbm.at[ch]
        pltpu.sync_copy(g_chunk.at[rows], acc_sh.at[rows])
        plsc.subcore_barrier()

        def issue_g(j, b):
            pltpu.async_copy(g_chunk.at[src_v.at[j]], bufs.at[b], sem_g.at[b])

        def wait_g(j, b):
            pltpu.make_async_copy(
                g_chunk.at[src_v.at[j]], bufs.at[b], sem_g.at[b]).wait()

        def issue_s(j, b):
            pltpu.async_copy(bufs.at[b], acc_sh.at[dst_v.at[j]], sem_s.at[b],
                             add=True)

        def wait_s(j, b):
            pltpu.make_async_copy(
                bufs.at[b], acc_sh.at[dst_v.at[j]], sem_s.at[b]).wait()

        for h in range(NH):
            pltpu.sync_copy(src_hbm.at[s].at[h], src_v)
            pltpu.sync_copy(dst_hbm.at[s].at[h], dst_v)
            for b in range(NBUF - 1):
                issue_g(b, b)
            for j in range(NBUF):
                wait_g(j, j % NBUF)
                issue_s(j, j % NBUF)
                if j >= 1:
                    wait_s(j - 1, (j - 1) % NBUF)
                issue_g(j + NBUF - 1, (j + NBUF - 1) % NBUF)

            def slots(j2, carry):
                for b in range(NBUF):
                    j = j2 * NBUF + b
                    wait_g(j, b)
                    issue_s(j, b)
                    wait_s(j - 1, (b + NBUF - 1) % NBUF)
                    issue_g(j + NBUF - 1, (b + NBUF - 1) % NBUF)
                return carry

            lax.fori_loop(1, NB // NBUF - 1, slots, 0)

            for j in range(NB - NBUF, NB):
                wait_g(j, j % NBUF)
                issue_s(j, j % NBUF)
                if j + NBUF - 1 < NB:
                    wait_s(j - 1, (j - 1) % NBUF)
                    issue_g(j + NBUF - 1, (j + NBUF - 1) % NBUF)
            for j in range(NB - NBUF, NB):
                wait_s(j, j % NBUF)
        plsc.subcore_barrier()
        pltpu.sync_copy(acc_sh.at[rows], out_hbm.at[ch].at[rows])
        plsc.subcore_barrier()


def _make_agg(nch):
    @jax.jit
    def call(src3, dst3, g):
        return pl.kernel(
            functools.partial(_agg_body, nch=nch),
            out_type=jax.ShapeDtypeStruct((nch, NP, CH), jnp.float32),
            mesh=plsc.VectorSubcoreMesh(**_MESH),
            scratch_types=[
                pltpu.VMEM((NB, EB), jnp.int32),
                pltpu.VMEM((NB, EB), jnp.int32),
                pltpu.VMEM((NBUF, EB, CH), jnp.float32),
                pltpu.VMEM_SHARED((NP, CH), jnp.float32),
                pltpu.SemaphoreType.DMA((NBUF,)),
                pltpu.SemaphoreType.DMA((NBUF,)),
            ],
        )(src3, dst3, g)

    return call


_agg2_call = _make_agg(2)
_agg4_call = _make_agg(4)


def _pre_body(deg_ref, x_ref, g_ref):
    d = deg_ref[0] + deg_ref[1] + 1.0
    dinv = lax.rsqrt(d)[:, None]
    g_ref[0] = x_ref[:, :CH] * dinv
    g_ref[1] = x_ref[:, CH:] * dinv


@jax.jit
def _pre_call(deg2, x):
    return pl.pallas_call(
        _pre_body,
        grid=(GI,),
        in_specs=[
            pl.BlockSpec((NC, R), lambda i: (0, i)),
            pl.BlockSpec((R, IN_DIM), lambda i: (i, 0)),
        ],
        out_specs=pl.BlockSpec((2, R, CH), lambda i: (0, i, 0)),
        out_shape=jax.ShapeDtypeStruct((2, NP, CH), jnp.float32),
    )(deg2, x)


def _layer_body(acc_ref, w_ref, deg_ref, b_ref, out_ref, *, kc, oc, emit_g):
    m = jnp.dot(acc_ref[0], w_ref[0], preferred_element_type=jnp.float32)
    for k in range(1, kc):
        m += jnp.dot(acc_ref[k], w_ref[k], preferred_element_type=jnp.float32)
    d = deg_ref[0] + deg_ref[1] + 1.0
    dinv = lax.rsqrt(d)[:, None]
    h = jnp.maximum(dinv * m + b_ref[0], 0.0)
    if emit_g:
        h = dinv * h
        for c in range(oc):
            out_ref[c] = h[:, c * CH:(c + 1) * CH]
    else:
        out_ref[...] = h


def _make_layer(kc, oc, emit_g):
    out_shape = (jax.ShapeDtypeStruct((oc, NP, CH), jnp.float32) if emit_g
                 else jax.ShapeDtypeStruct((N, oc * CH), jnp.float32))
    out_spec = (pl.BlockSpec((oc, R, CH), lambda i: (0, i, 0)) if emit_g
                else pl.BlockSpec((R, oc * CH), lambda i: (i, 0)))

    @jax.jit
    def call(acc, wr, deg2, br):
        return pl.pallas_call(
            functools.partial(_layer_body, kc=kc, oc=oc, emit_g=emit_g),
            grid=(GI,),
            in_specs=[
                pl.BlockSpec((kc, R, CH), lambda i: (0, i, 0)),
                pl.BlockSpec((kc, CH, oc * CH), lambda i: (0, 0, 0)),
                pl.BlockSpec((NC, R), lambda i: (0, i)),
                pl.BlockSpec((1, oc * CH), lambda i: (0, 0)),
            ],
            out_specs=out_spec,
            out_shape=out_shape,
        )(acc, wr, deg2, br)

    return call


_l1_call = _make_layer(kc=2, oc=4, emit_g=True)
_l2_call = _make_layer(kc=4, oc=4, emit_g=False)


def kernel(x, edge_index, W1, b1, W2, b2):
    src = jnp.pad(edge_index[0].astype(jnp.int32), (0, EP - E),
                  constant_values=NP - 1)
    dst = jnp.pad(edge_index[1].astype(jnp.int32), (0, EP - E),
                  constant_values=NP - 1)
    dst_deg = dst.reshape(NC * NS, -1, EB)
    src_agg = src.reshape(NS, NH, -1, EB)
    dst_agg = dst.reshape(NS, NH, -1, EB)

    ones = jnp.ones((EB, CH), jnp.float32)
    zeros = jnp.zeros((ROWS_T, CH), jnp.float32)

    degp = _deg_call(dst_deg, ones, zeros)
    deg2 = degp[:, :, 0]

    g1 = _pre_call(deg2, x)
    a1 = _agg2_call(src_agg, dst_agg, g1)
    g2 = _l1_call(a1, W1.reshape(2, CH, HID), deg2,
                  b1.reshape(1, HID))
    a2 = _agg4_call(src_agg, dst_agg, g2)
    out = _l2_call(a2, W2.reshape(4, CH, HID), deg2,
                   b2.reshape(1, HID))
    return out


# speedup vs baseline: 1.8822x; 1.8822x over previous
import functools

import jax
import jax.numpy as jnp
from jax import lax
from jax.experimental import pallas as pl
from jax.experimental.pallas import tpu as pltpu
from jax.experimental.pallas import tpu_sc as plsc

N = 10000
NP = 10240
E = 160000
IN_DIM = 256
HID = 512
CH = 128

NC = 2
NS = 16
EB = 100
EP = 160000
ROWS_T = NP // NS

R = 512
GI = NP // R

_MESH = dict(core_axis_name="c", subcore_axis_name="s", num_cores=NC,
             num_subcores=NS)


def _deg_body(dst_hbm, ones_hbm, zeros_hbm, out_hbm, idx_v, ones_v, acc_sh):
    c = lax.axis_index("c")
    s = lax.axis_index("s")
    w = c * NS + s
    pltpu.sync_copy(ones_hbm, ones_v)
    pltpu.sync_copy(dst_hbm.at[w], idx_v)
    rows = pl.ds(s * ROWS_T, ROWS_T)
    pltpu.sync_copy(zeros_hbm, acc_sh.at[rows])
    plsc.subcore_barrier()

    def step(j, carry):
        pltpu.sync_copy(ones_v, acc_sh.at[idx_v.at[j]], add=True)
        return carry

    lax.fori_loop(0, EP // (NC * NS * EB), step, 0)
    plsc.subcore_barrier()
    pltpu.sync_copy(acc_sh.at[rows], out_hbm.at[c].at[rows])


@jax.jit
def _deg_call(dst4, ones, zeros):
    return pl.kernel(
        _deg_body,
        out_type=jax.ShapeDtypeStruct((NC, NP, CH), jnp.float32),
        mesh=plsc.VectorSubcoreMesh(**_MESH),
        scratch_types=[
            pltpu.VMEM((EP // (NC * NS * EB), EB), jnp.int32),
            pltpu.VMEM((EB, CH), jnp.float32),
            pltpu.VMEM_SHARED((NP, CH), jnp.float32),
        ],
    )(dst4, ones, zeros)


NBUF = 2
NH = 2
NB = EP // (NS * NH * EB)


def _agg_body(src_hbm, dst_hbm, g_hbm, out_hbm, src_v, dst_v, bufs, acc_sh,
              sem_g, sem_s, *, nch):
    c = lax.axis_index("c")
    s = lax.axis_index("s")
    per_core = nch // NC
    rows = pl.ds(s * ROWS_T, ROWS_T)
    for k in range(per_core):
        ch = c * per_core + k
        g_chunk = g_hbm.at[ch]
        pltpu.sync_copy(g_chunk.at[rows], acc_sh.at[rows])
        plsc.subcore_barrier()

        def issue_g(j, b):
            pltpu.async_copy(g_chunk.at[src_v.at[j]], bufs.at[b], sem_g.at[b])

        def wait_g(j, b):
            pltpu.make_async_copy(
                g_chunk.at[src_v.at[j]], bufs.at[b], sem_g.at[b]).wait()

        def issue_s(j, b):
            pltpu.async_copy(bufs.at[b], acc_sh.at[dst_v.at[j]], sem_s.at[b],
                             add=True)

        def wait_s(j, b):
            pltpu.make_async_copy(
                bufs.at[b], acc_sh.at[dst_v.at[j]], sem_s.at[b]).wait()

        for h in range(NH):
            pltpu.sync_copy(src_hbm.at[s].at[h], src_v)
            pltpu.sync_copy(dst_hbm.at[s].at[h], dst_v)
            for b in range(NBUF - 1):
                issue_g(b, b)
            for j in range(NBUF):
                wait_g(j, j % NBUF)
                issue_s(j, j % NBUF)
                if j >= 1:
                    wait_s(j - 1, (j - 1) % NBUF)
                issue_g(j + NBUF - 1, (j + NBUF - 1) % NBUF)

            def slots(j2, carry):
                for b in range(NBUF):
                    j = j2 * NBUF + b
                    wait_g(j, b)
                    issue_s(j, b)
                    wait_s(j - 1, (b + NBUF - 1) % NBUF)
                    issue_g(j + NBUF - 1, (b + NBUF - 1) % NBUF)
                return carry

            lax.fori_loop(1, NB // NBUF - 1, slots, 0)

            for j in range(NB - NBUF, NB):
                wait_g(j, j % NBUF)
                issue_s(j, j % NBUF)
                if j + NBUF - 1 < NB:
                    wait_s(j - 1, (j - 1) % NBUF)
                    issue_g(j + NBUF - 1, (j + NBUF - 1) % NBUF)
            for j in range(NB - NBUF, NB):
                wait_s(j, j % NBUF)
        plsc.subcore_barrier()
        pltpu.sync_copy(acc_sh.at[rows], out_hbm.at[ch].at[rows])
        plsc.subcore_barrier()


def _make_agg(nch):
    @jax.jit
    def call(src3, dst3, g):
        return pl.kernel(
            functools.partial(_agg_body, nch=nch),
            out_type=jax.ShapeDtypeStruct((nch, NP, CH), jnp.float32),
            mesh=plsc.VectorSubcoreMesh(**_MESH),
            scratch_types=[
                pltpu.VMEM((NB, EB), jnp.int32),
                pltpu.VMEM((NB, EB), jnp.int32),
                pltpu.VMEM((NBUF, EB, CH), jnp.float32),
                pltpu.VMEM_SHARED((NP, CH), jnp.float32),
                pltpu.SemaphoreType.DMA((NBUF,)),
                pltpu.SemaphoreType.DMA((NBUF,)),
            ],
        )(src3, dst3, g)

    return call


_agg2_call = _make_agg(2)
_agg4_call = _make_agg(4)


def _pre_body(deg_ref, x_ref, g_ref):
    d = deg_ref[0] + deg_ref[1] + 1.0
    dinv = lax.rsqrt(d)[:, None]
    g_ref[0] = x_ref[:, :CH] * dinv
    g_ref[1] = x_ref[:, CH:] * dinv


@jax.jit
def _pre_call(deg2, x):
    return pl.pallas_call(
        _pre_body,
        grid=(GI,),
        in_specs=[
            pl.BlockSpec((NC, R), lambda i: (0, i)),
            pl.BlockSpec((R, IN_DIM), lambda i: (i, 0)),
        ],
        out_specs=pl.BlockSpec((2, R, CH), lambda i: (0, i, 0)),
        out_shape=jax.ShapeDtypeStruct((2, NP, CH), jnp.float32),
    )(deg2, x)


def _layer_body(acc_ref, w_ref, deg_ref, b_ref, out_ref, *, kc, oc, emit_g):
    m = jnp.dot(acc_ref[0], w_ref[0], preferred_element_type=jnp.float32)
    for k in range(1, kc):
        m += jnp.dot(acc_ref[k], w_ref[k], preferred_element_type=jnp.float32)
    d = deg_ref[0] + deg_ref[1] + 1.0
    dinv = lax.rsqrt(d)[:, None]
    h = jnp.maximum(dinv * m + b_ref[0], 0.0)
    if emit_g:
        h = dinv * h
        for c in range(oc):
            out_ref[c] = h[:, c * CH:(c + 1) * CH]
    else:
        out_ref[...] = h


def _make_layer(kc, oc, emit_g):
    out_shape = (jax.ShapeDtypeStruct((oc, NP, CH), jnp.float32) if emit_g
                 else jax.ShapeDtypeStruct((N, oc * CH), jnp.float32))
    out_spec = (pl.BlockSpec((oc, R, CH), lambda i: (0, i, 0)) if emit_g
                else pl.BlockSpec((R, oc * CH), lambda i: (i, 0)))

    @jax.jit
    def call(acc, wr, deg2, br):
        return pl.pallas_call(
            functools.partial(_layer_body, kc=kc, oc=oc, emit_g=emit_g),
            grid=(GI,),
            in_specs=[
                pl.BlockSpec((kc, R, CH), lambda i: (0, i, 0)),
                pl.BlockSpec((kc, CH, oc * CH), lambda i: (0, 0, 0)),
                pl.BlockSpec((NC, R), lambda i: (0, i)),
                pl.BlockSpec((1, oc * CH), lambda i: (0, 0)),
            ],
            out_specs=out_spec,
            out_shape=out_shape,
        )(acc, wr, deg2, br)

    return call


_l1_call = _make_layer(kc=2, oc=4, emit_g=True)
_l2_call = _make_layer(kc=4, oc=4, emit_g=False)


def kernel(x, edge_index, W1, b1, W2, b2):
    src = jnp.pad(edge_index[0].astype(jnp.int32), (0, EP - E),
                  constant_values=NP - 1)
    dst = jnp.pad(edge_index[1].astype(jnp.int32), (0, EP - E),
                  constant_values=NP - 1)
    dst_deg = dst.reshape(NC * NS, -1, EB)
    src_agg = src.reshape(NS, NH, -1, EB)
    dst_agg = dst.reshape(NS, NH, -1, EB)

    ones = jnp.ones((EB, CH), jnp.float32)
    zeros = jnp.zeros((ROWS_T, CH), jnp.float32)

    degp = _deg_call(dst_deg, ones, zeros)
    deg2 = degp[:, :, 0]

    g1 = _pre_call(deg2, x)
    a1 = _agg2_call(src_agg, dst_agg, g1)
    g2 = _l1_call(a1, W1.reshape(2, CH, HID), deg2,
                  b1.reshape(1, HID))
    a2 = _agg4_call(src_agg, dst_agg, g2)
    out = _l2_call(a2, W2.reshape(4, CH, HID), deg2,
                   b2.reshape(1, HID))
    return out


# speedup vs baseline: 2.0050x; 1.0653x over previous
import functools

import jax
import jax.numpy as jnp
from jax import lax
from jax.experimental import pallas as pl
from jax.experimental.pallas import tpu as pltpu
from jax.experimental.pallas import tpu_sc as plsc

N = 10000
NP = 10240
E = 160000
IN_DIM = 256
HID = 512
CH = 128

NC = 2
NS = 16
EB = 125
EP = 160000
ROWS_T = NP // NS

R = 512
GI = NP // R

_MESH = dict(core_axis_name="c", subcore_axis_name="s", num_cores=NC,
             num_subcores=NS)


def _deg_body(dst_hbm, ones_hbm, zeros_hbm, out_hbm, idx_v, ones_v, acc_sh):
    c = lax.axis_index("c")
    s = lax.axis_index("s")
    w = c * NS + s
    pltpu.sync_copy(ones_hbm, ones_v)
    pltpu.sync_copy(dst_hbm.at[w], idx_v)
    rows = pl.ds(s * ROWS_T, ROWS_T)
    pltpu.sync_copy(zeros_hbm, acc_sh.at[rows])
    plsc.subcore_barrier()

    def step(j, carry):
        pltpu.sync_copy(ones_v, acc_sh.at[idx_v.at[j]], add=True)
        return carry

    lax.fori_loop(0, EP // (NC * NS * EB), step, 0)
    plsc.subcore_barrier()
    pltpu.sync_copy(acc_sh.at[rows], out_hbm.at[c].at[rows])


@jax.jit
def _deg_call(dst4, ones, zeros):
    return pl.kernel(
        _deg_body,
        out_type=jax.ShapeDtypeStruct((NC, NP, CH), jnp.float32),
        mesh=plsc.VectorSubcoreMesh(**_MESH),
        scratch_types=[
            pltpu.VMEM((EP // (NC * NS * EB), EB), jnp.int32),
            pltpu.VMEM((EB, CH), jnp.float32),
            pltpu.VMEM_SHARED((NP, CH), jnp.float32),
        ],
    )(dst4, ones, zeros)


NBUF = 2
NH = 2
NB = EP // (NS * NH * EB)


def _agg_body(src_hbm, dst_hbm, g_hbm, out_hbm, src_v, dst_v, bufs, acc_sh,
              sem_g, sem_s, *, nch):
    c = lax.axis_index("c")
    s = lax.axis_index("s")
    per_core = nch // NC
    rows = pl.ds(s * ROWS_T, ROWS_T)
    for k in range(per_core):
        ch = c * per_core + k
        g_chunk = g_hbm.at[ch]
        pltpu.sync_copy(g_chunk.at[rows], acc_sh.at[rows])
        plsc.subcore_barrier()

        def issue_g(j, b):
            pltpu.async_copy(g_chunk.at[src_v.at[j]], bufs.at[b], sem_g.at[b])

        def wait_g(j, b):
            pltpu.make_async_copy(
                g_chunk.at[src_v.at[j]], bufs.at[b], sem_g.at[b]).wait()

        def issue_s(j, b):
            pltpu.async_copy(bufs.at[b], acc_sh.at[dst_v.at[j]], sem_s.at[b],
                             add=True)

        def wait_s(j, b):
            pltpu.make_async_copy(
                bufs.at[b], acc_sh.at[dst_v.at[j]], sem_s.at[b]).wait()

        for h in range(NH):
            pltpu.sync_copy(src_hbm.at[s].at[h], src_v)
            pltpu.sync_copy(dst_hbm.at[s].at[h], dst_v)
            for b in range(NBUF - 1):
                issue_g(b, b)
            for j in range(NBUF):
                wait_g(j, j % NBUF)
                issue_s(j, j % NBUF)
                if j >= 1:
                    wait_s(j - 1, (j - 1) % NBUF)
                issue_g(j + NBUF - 1, (j + NBUF - 1) % NBUF)

            def slots(j2, carry):
                for b in range(NBUF):
                    j = j2 * NBUF + b
                    wait_g(j, b)
                    issue_s(j, b)
                    wait_s(j - 1, (b + NBUF - 1) % NBUF)
                    issue_g(j + NBUF - 1, (b + NBUF - 1) % NBUF)
                return carry

            lax.fori_loop(1, NB // NBUF - 1, slots, 0)

            for j in range(NB - NBUF, NB):
                wait_g(j, j % NBUF)
                issue_s(j, j % NBUF)
                if j + NBUF - 1 < NB:
                    wait_s(j - 1, (j - 1) % NBUF)
                    issue_g(j + NBUF - 1, (j + NBUF - 1) % NBUF)
            for j in range(NB - NBUF, NB):
                wait_s(j, j % NBUF)
        plsc.subcore_barrier()
        pltpu.sync_copy(acc_sh.at[rows], out_hbm.at[ch].at[rows])
        plsc.subcore_barrier()


def _make_agg(nch):
    @jax.jit
    def call(src3, dst3, g):
        return pl.kernel(
            functools.partial(_agg_body, nch=nch),
            out_type=jax.ShapeDtypeStruct((nch, NP, CH), jnp.float32),
            mesh=plsc.VectorSubcoreMesh(**_MESH),
            scratch_types=[
                pltpu.VMEM((NB, EB), jnp.int32),
                pltpu.VMEM((NB, EB), jnp.int32),
                pltpu.VMEM((NBUF, EB, CH), jnp.float32),
                pltpu.VMEM_SHARED((NP, CH), jnp.float32),
                pltpu.SemaphoreType.DMA((NBUF,)),
                pltpu.SemaphoreType.DMA((NBUF,)),
            ],
        )(src3, dst3, g)

    return call


_agg2_call = _make_agg(2)
_agg4_call = _make_agg(4)


def _pre_body(deg_ref, x_ref, g_ref):
    d = deg_ref[0] + deg_ref[1] + 1.0
    dinv = lax.rsqrt(d)[:, None]
    g_ref[0] = x_ref[:, :CH] * dinv
    g_ref[1] = x_ref[:, CH:] * dinv


@jax.jit
def _pre_call(deg2, x):
    return pl.pallas_call(
        _pre_body,
        grid=(GI,),
        in_specs=[
            pl.BlockSpec((NC, R), lambda i: (0, i)),
            pl.BlockSpec((R, IN_DIM), lambda i: (i, 0)),
        ],
        out_specs=pl.BlockSpec((2, R, CH), lambda i: (0, i, 0)),
        out_shape=jax.ShapeDtypeStruct((2, NP, CH), jnp.float32),
    )(deg2, x)


def _layer_body(acc_ref, w_ref, deg_ref, b_ref, out_ref, *, kc, oc, emit_g):
    m = jnp.dot(acc_ref[0], w_ref[0], preferred_element_type=jnp.float32)
    for k in range(1, kc):
        m += jnp.dot(acc_ref[k], w_ref[k], preferred_element_type=jnp.float32)
    d = deg_ref[0] + deg_ref[1] + 1.0
    dinv = lax.rsqrt(d)[:, None]
    h = jnp.maximum(dinv * m + b_ref[0], 0.0)
    if emit_g:
        h = dinv * h
        for c in range(oc):
            out_ref[c] = h[:, c * CH:(c + 1) * CH]
    else:
        out_ref[...] = h


def _make_layer(kc, oc, emit_g):
    out_shape = (jax.ShapeDtypeStruct((oc, NP, CH), jnp.float32) if emit_g
                 else jax.ShapeDtypeStruct((N, oc * CH), jnp.float32))
    out_spec = (pl.BlockSpec((oc, R, CH), lambda i: (0, i, 0)) if emit_g
                else pl.BlockSpec((R, oc * CH), lambda i: (i, 0)))

    @jax.jit
    def call(acc, wr, deg2, br):
        return pl.pallas_call(
            functools.partial(_layer_body, kc=kc, oc=oc, emit_g=emit_g),
            grid=(GI,),
            in_specs=[
                pl.BlockSpec((kc, R, CH), lambda i: (0, i, 0)),
                pl.BlockSpec((kc, CH, oc * CH), lambda i: (0, 0, 0)),
                pl.BlockSpec((NC, R), lambda i: (0, i)),
                pl.BlockSpec((1, oc * CH), lambda i: (0, 0)),
            ],
            out_specs=out_spec,
            out_shape=out_shape,
        )(acc, wr, deg2, br)

    return call


_l1_call = _make_layer(kc=2, oc=4, emit_g=True)
_l2_call = _make_layer(kc=4, oc=4, emit_g=False)


def kernel(x, edge_index, W1, b1, W2, b2):
    src = jnp.pad(edge_index[0].astype(jnp.int32), (0, EP - E),
                  constant_values=NP - 1)
    dst = jnp.pad(edge_index[1].astype(jnp.int32), (0, EP - E),
                  constant_values=NP - 1)
    dst_deg = dst.reshape(NC * NS, -1, EB)
    src_agg = src.reshape(NS, NH, -1, EB)
    dst_agg = dst.reshape(NS, NH, -1, EB)

    ones = jnp.ones((EB, CH), jnp.float32)
    zeros = jnp.zeros((ROWS_T, CH), jnp.float32)

    degp = _deg_call(dst_deg, ones, zeros)
    deg2 = degp[:, :, 0]

    g1 = _pre_call(deg2, x)
    a1 = _agg2_call(src_agg, dst_agg, g1)
    g2 = _l1_call(a1, W1.reshape(2, CH, HID), deg2,
                  b1.reshape(1, HID))
    a2 = _agg4_call(src_agg, dst_agg, g2)
    out = _l2_call(a2, W2.reshape(4, CH, HID), deg2,
                   b2.reshape(1, HID))
    return out


# speedup vs baseline: 2.3214x; 1.1578x over previous
import functools

import jax
import jax.numpy as jnp
from jax import lax
from jax.experimental import pallas as pl
from jax.experimental.pallas import tpu as pltpu
from jax.experimental.pallas import tpu_sc as plsc

N = 10000
NP = 10240
E = 160000
IN_DIM = 256
HID = 512
CH = 128

NC = 2
NS = 16
EB = 125
EP = 160000
ROWS_T = NP // NS

R = 512
GI = NP // R

_MESH = dict(core_axis_name="c", subcore_axis_name="s", num_cores=NC,
             num_subcores=NS)


DW = 128


def _deg_body(ei_hbm, ones_hbm, zeros_hbm, out_hbm, idx_v, ones_v, acc_sh):
    c = lax.axis_index("c")
    s = lax.axis_index("s")
    pltpu.sync_copy(ones_hbm, ones_v)
    pltpu.sync_copy(ei_hbm.at[1].at[s].at[c], idx_v)
    rows = pl.ds(s * ROWS_T, ROWS_T)
    pltpu.sync_copy(zeros_hbm, acc_sh.at[rows])
    plsc.subcore_barrier()

    def step(j, carry):
        pltpu.sync_copy(ones_v, acc_sh.at[idx_v.at[j]], add=True)
        return carry

    lax.fori_loop(0, EP // (NC * NS * EB), step, 0)
    plsc.subcore_barrier()
    pltpu.sync_copy(acc_sh.at[rows], out_hbm.at[c].at[rows])


@jax.jit
def _deg_call(ei, ones, zeros):
    return pl.kernel(
        _deg_body,
        out_type=jax.ShapeDtypeStruct((NC, NP, DW), jnp.float32),
        mesh=plsc.VectorSubcoreMesh(**_MESH),
        scratch_types=[
            pltpu.VMEM((EP // (NC * NS * EB), EB), jnp.int32),
            pltpu.VMEM((EB, DW), jnp.float32),
            pltpu.VMEM_SHARED((NP, DW), jnp.float32),
        ],
    )(ei, ones, zeros)


NBUF = 2
NH = 2
NB = EP // (NS * NH * EB)


def _agg_body(ei_hbm, g_hbm, out_hbm, src_v, dst_v, bufs, acc_sh,
              sem_g, sem_s, *, nch):
    c = lax.axis_index("c")
    s = lax.axis_index("s")
    per_core = nch // NC
    rows = pl.ds(s * ROWS_T, ROWS_T)
    for k in range(per_core):
        ch = c * per_core + k
        g_chunk = g_hbm.at[ch]
        pltpu.sync_copy(g_chunk.at[rows], acc_sh.at[rows])
        plsc.subcore_barrier()

        def issue_g(j, b):
            pltpu.async_copy(g_chunk.at[src_v.at[j]], bufs.at[b], sem_g.at[b])

        def wait_g(j, b):
            pltpu.make_async_copy(
                g_chunk.at[src_v.at[j]], bufs.at[b], sem_g.at[b]).wait()

        def issue_s(j, b):
            pltpu.async_copy(bufs.at[b], acc_sh.at[dst_v.at[j]], sem_s.at[b],
                             add=True)

        def wait_s(j, b):
            pltpu.make_async_copy(
                bufs.at[b], acc_sh.at[dst_v.at[j]], sem_s.at[b]).wait()

        for h in range(NH):
            pltpu.sync_copy(ei_hbm.at[0].at[s].at[h], src_v)
            pltpu.sync_copy(ei_hbm.at[1].at[s].at[h], dst_v)
            issue_g(0, 0)
            wait_g(0, 0); issue_s(0, 0); issue_g(1, 1)
            wait_s(0, 0); issue_g(2, 0); wait_g(1, 1); issue_s(1, 1)
            wait_s(1, 1); issue_g(3, 1); wait_g(2, 0); issue_s(2, 0)

            def slots(j2, carry):
                for u in range(2):
                    j = j2 * 2 + u + 3
                    b, bn = (u + 1) % 2, u % 2
                    wait_s(j - 1, bn)
                    issue_g(j + 1, bn)
                    wait_g(j, b)
                    issue_s(j, b)
                return carry

            lax.fori_loop(0, (NB - 4) // 2, slots, 0)

            j = NB - 1
            wait_s(j - 1, j % 2 ^ 1)
            wait_g(j, j % 2)
            issue_s(j, j % 2)
            wait_s(j, j % 2)
        plsc.subcore_barrier()
        pltpu.sync_copy(acc_sh.at[rows], out_hbm.at[ch].at[rows])
        plsc.subcore_barrier()


def _make_agg(nch):
    @jax.jit
    def call(ei, g):
        return pl.kernel(
            functools.partial(_agg_body, nch=nch),
            out_type=jax.ShapeDtypeStruct((nch, NP, CH), jnp.float32),
            mesh=plsc.VectorSubcoreMesh(**_MESH),
            scratch_types=[
                pltpu.VMEM((NB, EB), jnp.int32),
                pltpu.VMEM((NB, EB), jnp.int32),
                pltpu.VMEM((NBUF, EB, CH), jnp.float32),
                pltpu.VMEM_SHARED((NP, CH), jnp.float32),
                pltpu.SemaphoreType.DMA((NBUF,)),
                pltpu.SemaphoreType.DMA((NBUF,)),
            ],
        )(ei, g)

    return call


_agg2_call = _make_agg(2)
_agg4_call = _make_agg(4)


def _pre_body(deg_ref, x_ref, g_ref):
    d = deg_ref[0] + deg_ref[1] + 1.0
    dinv = lax.rsqrt(d)[:, None]
    g_ref[0] = x_ref[:, :CH] * dinv
    g_ref[1] = x_ref[:, CH:] * dinv


@jax.jit
def _pre_call(deg2, x):
    return pl.pallas_call(
        _pre_body,
        grid=(GI,),
        in_specs=[
            pl.BlockSpec((NC, R), lambda i: (0, i)),
            pl.BlockSpec((R, IN_DIM), lambda i: (i, 0)),
        ],
        out_specs=pl.BlockSpec((2, R, CH), lambda i: (0, i, 0)),
        out_shape=jax.ShapeDtypeStruct((2, NP, CH), jnp.float32),
    )(deg2, x)


def _layer_body(acc_ref, w_ref, deg_ref, b_ref, out_ref, *, kc, oc, emit_g):
    m = jnp.dot(acc_ref[0], w_ref[0], preferred_element_type=jnp.float32)
    for k in range(1, kc):
        m += jnp.dot(acc_ref[k], w_ref[k], preferred_element_type=jnp.float32)
    d = deg_ref[0] + deg_ref[1] + 1.0
    dinv = lax.rsqrt(d)[:, None]
    h = jnp.maximum(dinv * m + b_ref[0], 0.0)
    if emit_g:
        h = dinv * h
        for c in range(oc):
            out_ref[c] = h[:, c * CH:(c + 1) * CH]
    else:
        out_ref[...] = h


def _make_layer(kc, oc, emit_g):
    out_shape = (jax.ShapeDtypeStruct((oc, NP, CH), jnp.float32) if emit_g
                 else jax.ShapeDtypeStruct((N, oc * CH), jnp.float32))
    out_spec = (pl.BlockSpec((oc, R, CH), lambda i: (0, i, 0)) if emit_g
                else pl.BlockSpec((R, oc * CH), lambda i: (i, 0)))

    @jax.jit
    def call(acc, wr, deg2, br):
        return pl.pallas_call(
            functools.partial(_layer_body, kc=kc, oc=oc, emit_g=emit_g),
            grid=(GI,),
            in_specs=[
                pl.BlockSpec((kc, R, CH), lambda i: (0, i, 0)),
                pl.BlockSpec((kc, CH, oc * CH), lambda i: (0, 0, 0)),
                pl.BlockSpec((NC, R), lambda i: (0, i)),
                pl.BlockSpec((1, oc * CH), lambda i: (0, 0)),
            ],
            out_specs=out_spec,
            out_shape=out_shape,
        )(acc, wr, deg2, br)

    return call


_l1_call = _make_layer(kc=2, oc=4, emit_g=True)
_l2_call = _make_layer(kc=4, oc=4, emit_g=False)


def kernel(x, edge_index, W1, b1, W2, b2):
    ei = jnp.pad(edge_index.astype(jnp.int32), ((0, 0), (0, EP - E)),
                 constant_values=NP - 1).reshape(2, NS, NH, NB, EB)

    ones = jnp.ones((EB, DW), jnp.float32)
    zeros = jnp.zeros((ROWS_T, DW), jnp.float32)

    degp = _deg_call(ei, ones, zeros)
    deg2 = degp[:, :, 0]

    g1 = _pre_call(deg2, x)
    a1 = _agg2_call(ei, g1)
    g2 = _l1_call(a1, W1.reshape(2, CH, HID), deg2,
                  b1.reshape(1, HID))
    a2 = _agg4_call(ei, g2)
    out = _l2_call(a2, W2.reshape(4, CH, HID), deg2,
                   b2.reshape(1, HID))
    return out


# speedup vs baseline: 2.3295x; 1.0035x over previous
import functools

import jax
import jax.numpy as jnp
from jax import lax
from jax.experimental import pallas as pl
from jax.experimental.pallas import tpu as pltpu
from jax.experimental.pallas import tpu_sc as plsc

N = 10000
NP = 10240
E = 160000
IN_DIM = 256
HID = 512
CH = 128

NC = 2
NS = 16
EB = 125
EP = 160000
ROWS_T = NP // NS

R = 512
GI = NP // R

_MESH = dict(core_axis_name="c", subcore_axis_name="s", num_cores=NC,
             num_subcores=NS)


DW = 128


def _deg_body(ei_hbm, ones_hbm, zeros_hbm, out_hbm, idx_v, ones_v, acc_sh,
              sem):
    c = lax.axis_index("c")
    s = lax.axis_index("s")
    pltpu.sync_copy(ones_hbm, ones_v)
    pltpu.sync_copy(ei_hbm.at[1].at[s].at[c], idx_v)
    rows = pl.ds(s * ROWS_T, ROWS_T)
    pltpu.sync_copy(zeros_hbm, acc_sh.at[rows])
    plsc.subcore_barrier()

    def step(j, carry):
        pltpu.async_copy(ones_v, acc_sh.at[idx_v.at[j]], sem, add=True)
        return carry

    lax.fori_loop(0, EP // (NC * NS * EB), step, 0)

    def drain(j, carry):
        pltpu.make_async_copy(ones_v, acc_sh.at[idx_v.at[j]], sem).wait()
        return carry

    lax.fori_loop(0, EP // (NC * NS * EB), drain, 0)
    plsc.subcore_barrier()
    pltpu.sync_copy(acc_sh.at[rows], out_hbm.at[c].at[rows])


@jax.jit
def _deg_call(ei, ones, zeros):
    return pl.kernel(
        _deg_body,
        out_type=jax.ShapeDtypeStruct((NC, NP, DW), jnp.float32),
        mesh=plsc.VectorSubcoreMesh(**_MESH),
        scratch_types=[
            pltpu.VMEM((EP // (NC * NS * EB), EB), jnp.int32),
            pltpu.VMEM((EB, DW), jnp.float32),
            pltpu.VMEM_SHARED((NP, DW), jnp.float32),
            pltpu.SemaphoreType.DMA,
        ],
    )(ei, ones, zeros)


NBUF = 2
NH = 2
NB = EP // (NS * NH * EB)


def _agg_body(ei_hbm, g_hbm, out_hbm, src_v, dst_v, bufs, acc_sh,
              sem_g, sem_s, *, nch):
    c = lax.axis_index("c")
    s = lax.axis_index("s")
    per_core = nch // NC
    rows = pl.ds(s * ROWS_T, ROWS_T)
    for k in range(per_core):
        ch = c * per_core + k
        g_chunk = g_hbm.at[ch]
        pltpu.sync_copy(g_chunk.at[rows], acc_sh.at[rows])
        plsc.subcore_barrier()

        def issue_g(j, b):
            pltpu.async_copy(g_chunk.at[src_v.at[j]], bufs.at[b], sem_g.at[b])

        def wait_g(j, b):
            pltpu.make_async_copy(
                g_chunk.at[src_v.at[j]], bufs.at[b], sem_g.at[b]).wait()

        def issue_s(j, b):
            pltpu.async_copy(bufs.at[b], acc_sh.at[dst_v.at[j]], sem_s.at[b],
                             add=True)

        def wait_s(j, b):
            pltpu.make_async_copy(
                bufs.at[b], acc_sh.at[dst_v.at[j]], sem_s.at[b]).wait()

        for h in range(NH):
            pltpu.sync_copy(ei_hbm.at[0].at[s].at[h], src_v)
            pltpu.sync_copy(ei_hbm.at[1].at[s].at[h], dst_v)
            issue_g(0, 0)
            wait_g(0, 0); issue_s(0, 0); issue_g(1, 1)
            wait_s(0, 0); issue_g(2, 0); wait_g(1, 1); issue_s(1, 1)
            wait_s(1, 1); issue_g(3, 1); wait_g(2, 0); issue_s(2, 0)

            def slots(j2, carry):
                for u in range(2):
                    j = j2 * 2 + u + 3
                    b, bn = (u + 1) % 2, u % 2
                    wait_s(j - 1, bn)
                    issue_g(j + 1, bn)
                    wait_g(j, b)
                    issue_s(j, b)
                return carry

            lax.fori_loop(0, (NB - 4) // 2, slots, 0)

            j = NB - 1
            wait_s(j - 1, j % 2 ^ 1)
            wait_g(j, j % 2)
            issue_s(j, j % 2)
            wait_s(j, j % 2)
        plsc.subcore_barrier()
        pltpu.sync_copy(acc_sh.at[rows], out_hbm.at[ch].at[rows])
        plsc.subcore_barrier()


def _make_agg(nch):
    @jax.jit
    def call(ei, g):
        return pl.kernel(
            functools.partial(_agg_body, nch=nch),
            out_type=jax.ShapeDtypeStruct((nch, NP, CH), jnp.float32),
            mesh=plsc.VectorSubcoreMesh(**_MESH),
            scratch_types=[
                pltpu.VMEM((NB, EB), jnp.int32),
                pltpu.VMEM((NB, EB), jnp.int32),
                pltpu.VMEM((NBUF, EB, CH), jnp.float32),
                pltpu.VMEM_SHARED((NP, CH), jnp.float32),
                pltpu.SemaphoreType.DMA((NBUF,)),
                pltpu.SemaphoreType.DMA((NBUF,)),
            ],
        )(ei, g)

    return call


_agg2_call = _make_agg(2)
_agg4_call = _make_agg(4)


def _pre_body(deg_ref, x_ref, g_ref):
    d = deg_ref[0] + deg_ref[1] + 1.0
    dinv = lax.rsqrt(d)[:, None]
    g_ref[0] = x_ref[:, :CH] * dinv
    g_ref[1] = x_ref[:, CH:] * dinv


@jax.jit
def _pre_call(deg2, x):
    return pl.pallas_call(
        _pre_body,
        grid=(GI,),
        in_specs=[
            pl.BlockSpec((NC, R), lambda i: (0, i)),
            pl.BlockSpec((R, IN_DIM), lambda i: (i, 0)),
        ],
        out_specs=pl.BlockSpec((2, R, CH), lambda i: (0, i, 0)),
        out_shape=jax.ShapeDtypeStruct((2, NP, CH), jnp.float32),
    )(deg2, x)


def _layer_body(acc_ref, w_ref, deg_ref, b_ref, out_ref, *, kc, oc, emit_g):
    m = jnp.dot(acc_ref[0], w_ref[0], preferred_element_type=jnp.float32)
    for k in range(1, kc):
        m += jnp.dot(acc_ref[k], w_ref[k], preferred_element_type=jnp.float32)
    d = deg_ref[0] + deg_ref[1] + 1.0
    dinv = lax.rsqrt(d)[:, None]
    h = jnp.maximum(dinv * m + b_ref[0], 0.0)
    if emit_g:
        h = dinv * h
        for c in range(oc):
            out_ref[c] = h[:, c * CH:(c + 1) * CH]
    else:
        out_ref[...] = h


def _make_layer(kc, oc, emit_g):
    out_shape = (jax.ShapeDtypeStruct((oc, NP, CH), jnp.float32) if emit_g
                 else jax.ShapeDtypeStruct((N, oc * CH), jnp.float32))
    out_spec = (pl.BlockSpec((oc, R, CH), lambda i: (0, i, 0)) if emit_g
                else pl.BlockSpec((R, oc * CH), lambda i: (i, 0)))

    @jax.jit
    def call(acc, wr, deg2, br):
        return pl.pallas_call(
            functools.partial(_layer_body, kc=kc, oc=oc, emit_g=emit_g),
            grid=(GI,),
            in_specs=[
                pl.BlockSpec((kc, R, CH), lambda i: (0, i, 0)),
                pl.BlockSpec((kc, CH, oc * CH), lambda i: (0, 0, 0)),
                pl.BlockSpec((NC, R), lambda i: (0, i)),
                pl.BlockSpec((1, oc * CH), lambda i: (0, 0)),
            ],
            out_specs=out_spec,
            out_shape=out_shape,
        )(acc, wr, deg2, br)

    return call


_l1_call = _make_layer(kc=2, oc=4, emit_g=True)
_l2_call = _make_layer(kc=4, oc=4, emit_g=False)


def kernel(x, edge_index, W1, b1, W2, b2):
    ei = jnp.pad(edge_index.astype(jnp.int32), ((0, 0), (0, EP - E)),
                 constant_values=NP - 1).reshape(2, NS, NH, NB, EB)

    ones = jnp.ones((EB, DW), jnp.float32)
    zeros = jnp.zeros((ROWS_T, DW), jnp.float32)

    degp = _deg_call(ei, ones, zeros)
    deg2 = degp[:, :, 0]

    g1 = _pre_call(deg2, x)
    a1 = _agg2_call(ei, g1)
    g2 = _l1_call(a1, W1.reshape(2, CH, HID), deg2,
                  b1.reshape(1, HID))
    a2 = _agg4_call(ei, g2)
    out = _l2_call(a2, W2.reshape(4, CH, HID), deg2,
                   b2.reshape(1, HID))
    return out
